# Initial kernel scaffold; baseline (speedup 1.0000x reference)
import jax
import jax.numpy as jnp
from jax.experimental import pallas as pl


def kernel(raw_concept_ids, concept_ids, value_ids, position_ids, delta_ts, abs_ts, patient_contexts, raw_tab, conc_tab, val_tab, pos_tab, rel_lw, rel_lb, rel_fw, rel_fb, abs_lw, abs_lb, abs_fw, abs_fb, time_w, ctx_token, ctx_w, final_w, final_b, ln_g, ln_b):
    raise NotImplementedError("write your pallas kernel here")


# speedup vs baseline: 3.3431x; 3.3431x over previous
import functools
import math

import jax
import jax.numpy as jnp
from jax import lax
from jax.experimental import pallas as pl
from jax.experimental.pallas import tpu as pltpu
from jax.experimental.pallas import tpu_sc as plsc

B, T, D = 1024, 200, 128
CTX = 64
BT = B * T

NC, NS = 2, 16
NW = NC * NS
RPW = BT // NW
CHUNK = 128


def _transform_body(tab_ref, w_ref, out_ref):
    out_ref[...] = lax.dot_general(
        tab_ref[...], w_ref[...], (((1,), (1,)), ((), ())),
        preferred_element_type=jnp.float32)


def _transform(tab, final_w, seg):
    v = tab.shape[0]
    rb = 1000
    return pl.pallas_call(
        _transform_body,
        grid=(v // rb,),
        in_specs=[
            pl.BlockSpec((rb, D), lambda i: (i, 0)),
            pl.BlockSpec((D, D), lambda i, s=seg: (0, s)),
        ],
        out_specs=pl.BlockSpec((rb, D), lambda i: (i, 0)),
        out_shape=jax.ShapeDtypeStruct((v, D), jnp.float32),
    )(tab, final_w)


def _gather_sum_body(rid, cid, vid, pid, t0, t1, t2, t3, out,
                     idx0, idx1, idx2, idx3, b0, b1, b2, b3, acc, sem):
    wid = lax.axis_index("s") * NC + lax.axis_index("c")
    base = wid * RPW
    pltpu.sync_copy(rid.at[pl.ds(base, RPW)], idx0)
    pltpu.sync_copy(cid.at[pl.ds(base, RPW)], idx1)
    pltpu.sync_copy(vid.at[pl.ds(base, RPW)], idx2)
    pltpu.sync_copy(pid.at[pl.ds(base, RPW)], idx3)

    def chunk(ci, carry):
        off = ci * CHUNK
        d0 = pltpu.async_copy(t0.at[idx0.at[pl.ds(off, CHUNK)]], b0, sem)
        d1 = pltpu.async_copy(t1.at[idx1.at[pl.ds(off, CHUNK)]], b1, sem)
        d2 = pltpu.async_copy(t2.at[idx2.at[pl.ds(off, CHUNK)]], b2, sem)
        d3 = pltpu.async_copy(t3.at[idx3.at[pl.ds(off, CHUNK)]], b3, sem)
        d0.wait()
        d1.wait()
        d2.wait()
        d3.wait()

        def row(r, c):
            for j in range(D // 16):
                sl = pl.ds(j * 16, 16)
                acc[r, sl] = ((b0[r, sl] + b1[r, sl])
                              + (b2[r, sl] + b3[r, sl]))
            return c

        lax.fori_loop(0, CHUNK, row, 0)
        pltpu.sync_copy(acc, out.at[pl.ds(base + off, CHUNK)])
        return carry

    lax.fori_loop(0, RPW // CHUNK, chunk, 0)


def _gather_sum(rid, cid, vid, pid, t0, t1, t2, t3):
    mesh = plsc.VectorSubcoreMesh(
        core_axis_name="c", subcore_axis_name="s",
        num_cores=NC, num_subcores=NS)
    f = functools.partial(
        pl.kernel, mesh=mesh,
        out_type=jax.ShapeDtypeStruct((BT, D), jnp.float32),
        scratch_types=[
            pltpu.VMEM((RPW,), jnp.int32),
            pltpu.VMEM((RPW,), jnp.int32),
            pltpu.VMEM((RPW,), jnp.int32),
            pltpu.VMEM((RPW,), jnp.int32),
            pltpu.VMEM((CHUNK, D), jnp.float32),
            pltpu.VMEM((CHUNK, D), jnp.float32),
            pltpu.VMEM((CHUNK, D), jnp.float32),
            pltpu.VMEM((CHUNK, D), jnp.float32),
            pltpu.VMEM((CHUNK, D), jnp.float32),
            pltpu.SemaphoreType.DMA,
        ],
    )(_gather_sum_body)
    return f(rid, cid, vid, pid, t0, t1, t2, t3)


NB = 8


def _finish_body(s_ref, dts_ref, ats_ref, pc_ref, time_w_ref, w4_ref,
                 fb_ref, ctok_ref, ctw_ref, w16_ref, b16_ref, g_ref, bb_ref,
                 out_ref):
    n = NB * T
    d = dts_ref[...]
    a = ats_ref[...]
    w16 = w16_ref[...]
    b16 = b16_ref[...]
    ph_r = d * w16[:, 0:8] + b16[:, 0:8]
    ph_a = a * w16[:, 8:16] + b16[:, 8:16]
    ph = jnp.concatenate([ph_r, ph_a], axis=1)
    lane = lax.broadcasted_iota(jnp.int32, (n, 16), 1)
    tcat = jnp.where(lane % 8 == 0, ph, jnp.sin(ph))
    m2 = lax.dot_general(w4_ref[...], time_w_ref[...],
                         (((1,), (0,)), ((), ())),
                         preferred_element_type=jnp.float32)
    t_part = lax.dot_general(tcat, m2, (((1,), (1,)), ((), ())),
                             preferred_element_type=jnp.float32)
    scale = 1.0 / math.sqrt(D)
    ev = (s_ref[...].reshape(n, D) + t_part + fb_ref[...]) * scale
    ctx = ctok_ref[...] + lax.dot_general(
        pc_ref[...], ctw_ref[...], (((1,), (1,)), ((), ())),
        preferred_element_type=jnp.float32)
    seq = jnp.concatenate([ctx.reshape(NB, 1, D), ev.reshape(NB, T, D)],
                          axis=1)
    mu = jnp.mean(seq, axis=-1, keepdims=True)
    var = jnp.mean((seq - mu) ** 2, axis=-1, keepdims=True)
    out_ref[...] = (seq - mu) * lax.rsqrt(var + 1e-5) * g_ref[...] + bb_ref[...]


def _finish(s3, dts, ats, pc, time_w, final_w, fb, ctok, ctw, w16, b16,
            ln_g3, ln_b3):
    return pl.pallas_call(
        _finish_body,
        grid=(B // NB,),
        in_specs=[
            pl.BlockSpec((NB, T, D), lambda i: (i, 0, 0)),
            pl.BlockSpec((NB * T, 1), lambda i: (i, 0)),
            pl.BlockSpec((NB * T, 1), lambda i: (i, 0)),
            pl.BlockSpec((NB, CTX), lambda i: (i, 0)),
            pl.BlockSpec((D, 16), lambda i: (0, 0)),
            pl.BlockSpec((D, D), lambda i: (0, 4)),
            pl.BlockSpec((1, D), lambda i: (0, 0)),
            pl.BlockSpec((1, D), lambda i: (0, 0)),
            pl.BlockSpec((D, CTX), lambda i: (0, 0)),
            pl.BlockSpec((1, 16), lambda i: (0, 0)),
            pl.BlockSpec((1, 16), lambda i: (0, 0)),
            pl.BlockSpec((1, 1, D), lambda i: (0, 0, 0)),
            pl.BlockSpec((1, 1, D), lambda i: (0, 0, 0)),
        ],
        out_specs=pl.BlockSpec((NB, T + 1, D), lambda i: (i, 0, 0)),
        out_shape=jax.ShapeDtypeStruct((B, T + 1, D), jnp.float32),
    )(s3, dts, ats, pc, time_w, final_w, fb, ctok, ctw, w16, b16,
      ln_g3, ln_b3)


def kernel(raw_concept_ids, concept_ids, value_ids, position_ids, delta_ts,
           abs_ts, patient_contexts, raw_tab, conc_tab, val_tab, pos_tab,
           rel_lw, rel_lb, rel_fw, rel_fb, abs_lw, abs_lb, abs_fw, abs_fb,
           time_w, ctx_token, ctx_w, final_w, final_b, ln_g, ln_b):
    rid = raw_concept_ids.reshape(BT)
    cid = concept_ids.reshape(BT)
    vid = value_ids.reshape(BT)
    pid = position_ids.reshape(BT)

    t_raw = _transform(raw_tab, final_w, 0)
    t_con = _transform(conc_tab, final_w, 1)
    t_val = _transform(val_tab, final_w, 2)
    t_pos = _transform(pos_tab, final_w, 3)

    s = _gather_sum(rid, cid, vid, pid, t_raw, t_con, t_val, t_pos)

    w16 = jnp.concatenate([rel_lw.reshape(1), rel_fw.reshape(7),
                           abs_lw.reshape(1), abs_fw.reshape(7)]).reshape(1, 16)
    b16 = jnp.concatenate([rel_lb, rel_fb, abs_lb, abs_fb]).reshape(1, 16)

    return _finish(s.reshape(B, T, D), delta_ts.reshape(BT, 1),
                   abs_ts.reshape(BT, 1), patient_contexts,
                   time_w, final_w, final_b.reshape(1, D),
                   ctx_token.reshape(1, D), ctx_w, w16, b16,
                   ln_g.reshape(1, 1, D), ln_b.reshape(1, 1, D))


# speedup vs baseline: 6.3511x; 1.8997x over previous
import functools
import math

import jax
import jax.numpy as jnp
from jax import lax
from jax.experimental import pallas as pl
from jax.experimental.pallas import tpu as pltpu
from jax.experimental.pallas import tpu_sc as plsc

B, T, D = 1024, 200, 128
CTX = 64
BT = B * T

NC, NS = 2, 16
NW = NC * NS
RPW = BT // NW
CHUNK = 64


def _transform_body(tab_ref, w_ref, out_ref):
    out_ref[...] = lax.dot_general(
        tab_ref[...], w_ref[...], (((1,), (1,)), ((), ())),
        preferred_element_type=jnp.float32)


def _transform(tab, final_w, seg):
    v = tab.shape[0]
    rb = 1000
    return pl.pallas_call(
        _transform_body,
        grid=(v // rb,),
        in_specs=[
            pl.BlockSpec((rb, D), lambda i: (i, 0)),
            pl.BlockSpec((D, D), lambda i, s=seg: (0, s)),
        ],
        out_specs=pl.BlockSpec((rb, D), lambda i: (i, 0)),
        out_shape=jax.ShapeDtypeStruct((v, D), jnp.float32),
    )(tab, final_w)


def _gather_sum_body(rid, cid, vid, pid, t0, t1, t2, t3, out,
                     idx0, idx1, idx2, idx3,
                     a0, a1, a2, a3, c0, c1, c2, c3,
                     accA, accB, gsemA, gsemB, ssemA, ssemB):
    wid = lax.axis_index("s") * NC + lax.axis_index("c")
    base = wid * RPW
    nch = RPW // CHUNK
    pltpu.sync_copy(rid.at[pl.ds(base, RPW)], idx0)
    pltpu.sync_copy(cid.at[pl.ds(base, RPW)], idx1)
    pltpu.sync_copy(vid.at[pl.ds(base, RPW)], idx2)
    pltpu.sync_copy(pid.at[pl.ds(base, RPW)], idx3)

    def fire(off, bufs, sem):
        pltpu.async_copy(t0.at[idx0.at[pl.ds(off, CHUNK)]], bufs[0], sem)
        pltpu.async_copy(t1.at[idx1.at[pl.ds(off, CHUNK)]], bufs[1], sem)
        pltpu.async_copy(t2.at[idx2.at[pl.ds(off, CHUNK)]], bufs[2], sem)
        pltpu.async_copy(t3.at[idx3.at[pl.ds(off, CHUNK)]], bufs[3], sem)

    def drain(bufs, sem):
        for b in bufs:
            pltpu.make_async_copy(
                t0.at[idx0.at[pl.ds(0, CHUNK)]], b, sem).wait()

    def summed(bufs, acc):
        def row(r, c):
            for j in range(D // 16):
                sl = pl.ds(j * 16, 16)
                acc[r, sl] = ((bufs[0][r, sl] + bufs[1][r, sl])
                              + (bufs[2][r, sl] + bufs[3][r, sl]))
            return c
        lax.fori_loop(0, CHUNK, row, 0)

    bufsA = (a0, a1, a2, a3)
    bufsB = (c0, c1, c2, c3)
    fire(0, bufsA, gsemA)

    def step(i, carry):
        offa = (2 * i) * CHUNK
        offb = offa + CHUNK
        fire(offb, bufsB, gsemB)
        drain(bufsA, gsemA)

        @pl.when(i > 0)
        def _():
            pltpu.make_async_copy(accA, out.at[pl.ds(base, CHUNK)],
                                  ssemA).wait()
        summed(bufsA, accA)
        pltpu.async_copy(accA, out.at[pl.ds(base + offa, CHUNK)], ssemA)

        @pl.when(i < nch // 2 - 1)
        def _():
            fire(offa + 2 * CHUNK, bufsA, gsemA)
        drain(bufsB, gsemB)

        @pl.when(i > 0)
        def _():
            pltpu.make_async_copy(accB, out.at[pl.ds(base, CHUNK)],
                                  ssemB).wait()
        summed(bufsB, accB)
        pltpu.async_copy(accB, out.at[pl.ds(base + offb, CHUNK)], ssemB)
        return carry

    lax.fori_loop(0, nch // 2, step, 0)
    pltpu.make_async_copy(accA, out.at[pl.ds(base, CHUNK)], ssemA).wait()
    pltpu.make_async_copy(accB, out.at[pl.ds(base, CHUNK)], ssemB).wait()


def _gather_sum(rid, cid, vid, pid, t0, t1, t2, t3):
    mesh = plsc.VectorSubcoreMesh(
        core_axis_name="c", subcore_axis_name="s",
        num_cores=NC, num_subcores=NS)
    buf = pltpu.VMEM((CHUNK, D), jnp.float32)
    f = functools.partial(
        pl.kernel, mesh=mesh,
        out_type=jax.ShapeDtypeStruct((BT, D), jnp.float32),
        scratch_types=[
            pltpu.VMEM((RPW,), jnp.int32),
            pltpu.VMEM((RPW,), jnp.int32),
            pltpu.VMEM((RPW,), jnp.int32),
            pltpu.VMEM((RPW,), jnp.int32),
            buf, buf, buf, buf, buf, buf, buf, buf, buf, buf,
            pltpu.SemaphoreType.DMA,
            pltpu.SemaphoreType.DMA,
            pltpu.SemaphoreType.DMA,
            pltpu.SemaphoreType.DMA,
        ],
    )(_gather_sum_body)
    return f(rid, cid, vid, pid, t0, t1, t2, t3)


NB = 16


def _finish_body(s_ref, dts_ref, ats_ref, pc_ref, time_w_ref, w4_ref,
                 fb_ref, ctok_ref, ctw_ref, w16_ref, b16_ref, g_ref, bb_ref,
                 out_ref):
    n = NB * T
    d = dts_ref[...].reshape(1, n)
    a = ats_ref[...].reshape(1, n)
    w16 = w16_ref[...]
    b16 = b16_ref[...]
    ph_r = w16[0:8, :] * d + b16[0:8, :]
    ph_a = w16[8:16, :] * a + b16[8:16, :]
    ph = jnp.concatenate([ph_r, ph_a], axis=0)
    row = lax.broadcasted_iota(jnp.int32, (16, n), 0)
    tcat = jnp.where(row % 8 == 0, ph, jnp.sin(ph))
    m2 = lax.dot_general(w4_ref[...], time_w_ref[...],
                         (((1,), (0,)), ((), ())),
                         preferred_element_type=jnp.float32)
    t_part = lax.dot_general(tcat, m2, (((0,), (1,)), ((), ())),
                             preferred_element_type=jnp.float32)
    scale = 1.0 / math.sqrt(D)
    ev = (s_ref[...].reshape(n, D) + t_part + fb_ref[...]) * scale
    ctx = ctok_ref[...] + lax.dot_general(
        pc_ref[...], ctw_ref[...], (((1,), (1,)), ((), ())),
        preferred_element_type=jnp.float32)
    seq = jnp.concatenate([ctx.reshape(NB, 1, D), ev.reshape(NB, T, D)],
                          axis=1)
    mu = jnp.mean(seq, axis=-1, keepdims=True)
    var = jnp.mean((seq - mu) ** 2, axis=-1, keepdims=True)
    out_ref[...] = (seq - mu) * lax.rsqrt(var + 1e-5) * g_ref[...] + bb_ref[...]


def _finish(s3, dts, ats, pc, time_w, final_w, fb, ctok, ctw, w16, b16,
            ln_g3, ln_b3):
    return pl.pallas_call(
        _finish_body,
        grid=(B // NB,),
        in_specs=[
            pl.BlockSpec((NB, T, D), lambda i: (i, 0, 0)),
            pl.BlockSpec((1, 1, NB * T), lambda i: (i, 0, 0)),
            pl.BlockSpec((1, 1, NB * T), lambda i: (i, 0, 0)),
            pl.BlockSpec((NB, CTX), lambda i: (i, 0)),
            pl.BlockSpec((D, 16), lambda i: (0, 0)),
            pl.BlockSpec((D, D), lambda i: (0, 4)),
            pl.BlockSpec((1, D), lambda i: (0, 0)),
            pl.BlockSpec((1, D), lambda i: (0, 0)),
            pl.BlockSpec((D, CTX), lambda i: (0, 0)),
            pl.BlockSpec((16, 1), lambda i: (0, 0)),
            pl.BlockSpec((16, 1), lambda i: (0, 0)),
            pl.BlockSpec((1, 1, D), lambda i: (0, 0, 0)),
            pl.BlockSpec((1, 1, D), lambda i: (0, 0, 0)),
        ],
        out_specs=pl.BlockSpec((NB, T + 1, D), lambda i: (i, 0, 0)),
        out_shape=jax.ShapeDtypeStruct((B, T + 1, D), jnp.float32),
    )(s3, dts, ats, pc, time_w, final_w, fb, ctok, ctw, w16, b16,
      ln_g3, ln_b3)


def kernel(raw_concept_ids, concept_ids, value_ids, position_ids, delta_ts,
           abs_ts, patient_contexts, raw_tab, conc_tab, val_tab, pos_tab,
           rel_lw, rel_lb, rel_fw, rel_fb, abs_lw, abs_lb, abs_fw, abs_fb,
           time_w, ctx_token, ctx_w, final_w, final_b, ln_g, ln_b):
    rid = raw_concept_ids.reshape(BT)
    cid = concept_ids.reshape(BT)
    vid = value_ids.reshape(BT)
    pid = position_ids.reshape(BT)

    t_raw = _transform(raw_tab, final_w, 0)
    t_con = _transform(conc_tab, final_w, 1)
    t_val = _transform(val_tab, final_w, 2)
    t_pos = _transform(pos_tab, final_w, 3)

    s = _gather_sum(rid, cid, vid, pid, t_raw, t_con, t_val, t_pos)

    w16 = jnp.concatenate([rel_lw.reshape(1), rel_fw.reshape(7),
                           abs_lw.reshape(1), abs_fw.reshape(7)]).reshape(16, 1)
    b16 = jnp.concatenate([rel_lb, rel_fb, abs_lb, abs_fb]).reshape(16, 1)

    return _finish(s.reshape(B, T, D), delta_ts.reshape(B // NB, 1, NB * T),
                   abs_ts.reshape(B // NB, 1, NB * T), patient_contexts,
                   time_w, final_w, final_b.reshape(1, D),
                   ctx_token.reshape(1, D), ctx_w, w16, b16,
                   ln_g.reshape(1, 1, D), ln_b.reshape(1, 1, D))


# speedup vs baseline: 8.9867x; 1.4150x over previous
import functools
import math

import jax
import jax.numpy as jnp
from jax import lax
from jax.experimental import pallas as pl
from jax.experimental.pallas import tpu as pltpu
from jax.experimental.pallas import tpu_sc as plsc

B, T, D = 1024, 200, 128
CTX = 64
BT = B * T

NC, NS = 2, 16
NW = NC * NS
RPW = BT // NW
CHUNK = 80


def _transform_body(tab_ref, w_ref, out_ref):
    out_ref[...] = lax.dot_general(
        tab_ref[...], w_ref[...], (((1,), (1,)), ((), ())),
        preferred_element_type=jnp.float32)


def _transform(tab, final_w, seg):
    v = tab.shape[0]
    rb = 5000 if v % 5000 == 0 else 1000
    return pl.pallas_call(
        _transform_body,
        grid=(v // rb,),
        in_specs=[
            pl.BlockSpec((rb, D), lambda i: (i, 0)),
            pl.BlockSpec((D, D), lambda i, s=seg: (0, s)),
        ],
        out_specs=pl.BlockSpec((rb, D), lambda i: (i, 0)),
        out_shape=jax.ShapeDtypeStruct((v, D), jnp.float32),
    )(tab, final_w)


def _gather_sum_body(rpw, rid, cid, vid, pid, t0, t1, t2, t3, out,
                     idx0, idx1, idx2, idx3,
                     a0, a1, a2, a3, c0, c1, c2, c3,
                     accA, accB, gsemA, gsemB, ssemA, ssemB):
    wid = lax.axis_index("s") * NC + lax.axis_index("c")
    base = wid * rpw
    nch = rpw // CHUNK
    pltpu.sync_copy(rid.at[pl.ds(base, rpw)], idx0)
    pltpu.sync_copy(cid.at[pl.ds(base, rpw)], idx1)
    pltpu.sync_copy(vid.at[pl.ds(base, rpw)], idx2)
    pltpu.sync_copy(pid.at[pl.ds(base, rpw)], idx3)

    def fire(off, bufs, sem):
        pltpu.async_copy(t0.at[idx0.at[pl.ds(off, CHUNK)]], bufs[0], sem)
        pltpu.async_copy(t1.at[idx1.at[pl.ds(off, CHUNK)]], bufs[1], sem)
        pltpu.async_copy(t2.at[idx2.at[pl.ds(off, CHUNK)]], bufs[2], sem)
        pltpu.async_copy(t3.at[idx3.at[pl.ds(off, CHUNK)]], bufs[3], sem)

    def drain(bufs, sem):
        for b in bufs:
            pltpu.make_async_copy(
                t0.at[idx0.at[pl.ds(0, CHUNK)]], b, sem).wait()

    def summed(bufs, acc):
        @plsc.parallel_loop(0, CHUNK, unroll=4)
        def _(r):
            for j in range(D // 16):
                sl = pl.ds(j * 16, 16)
                acc[r, sl] = ((bufs[0][r, sl] + bufs[1][r, sl])
                              + (bufs[2][r, sl] + bufs[3][r, sl]))

    bufsA = (a0, a1, a2, a3)
    bufsB = (c0, c1, c2, c3)
    fire(0, bufsA, gsemA)

    def step(i, carry):
        offa = (2 * i) * CHUNK
        offb = offa + CHUNK
        fire(offb, bufsB, gsemB)
        drain(bufsA, gsemA)

        @pl.when(i > 0)
        def _():
            pltpu.make_async_copy(accA, out.at[pl.ds(base, CHUNK)],
                                  ssemA).wait()
        summed(bufsA, accA)
        pltpu.async_copy(accA, out.at[pl.ds(base + offa, CHUNK)], ssemA)

        @pl.when(i < nch // 2 - 1)
        def _():
            fire(offa + 2 * CHUNK, bufsA, gsemA)
        drain(bufsB, gsemB)

        @pl.when(i > 0)
        def _():
            pltpu.make_async_copy(accB, out.at[pl.ds(base, CHUNK)],
                                  ssemB).wait()
        summed(bufsB, accB)
        pltpu.async_copy(accB, out.at[pl.ds(base + offb, CHUNK)], ssemB)
        return carry

    lax.fori_loop(0, nch // 2, step, 0)
    pltpu.make_async_copy(accA, out.at[pl.ds(base, CHUNK)], ssemA).wait()
    pltpu.make_async_copy(accB, out.at[pl.ds(base, CHUNK)], ssemB).wait()


def _gather_sum(rid, cid, vid, pid, t0, t1, t2, t3):
    rows = rid.shape[0]
    rpw = rows // NW
    mesh = plsc.VectorSubcoreMesh(
        core_axis_name="c", subcore_axis_name="s",
        num_cores=NC, num_subcores=NS)
    buf = pltpu.VMEM((CHUNK, D), jnp.float32)
    acc = pltpu.VMEM((CHUNK, D), jnp.float32)
    f = functools.partial(
        pl.kernel, mesh=mesh,
        out_type=jax.ShapeDtypeStruct((rows, D), jnp.float32),
        scratch_types=[
            pltpu.VMEM((rpw,), jnp.int32),
            pltpu.VMEM((rpw,), jnp.int32),
            pltpu.VMEM((rpw,), jnp.int32),
            pltpu.VMEM((rpw,), jnp.int32),
            buf, buf, buf, buf, buf, buf, buf, buf, acc, acc,
            pltpu.SemaphoreType.DMA,
            pltpu.SemaphoreType.DMA,
            pltpu.SemaphoreType.DMA,
            pltpu.SemaphoreType.DMA,
        ],
    )(functools.partial(_gather_sum_body, rpw))
    return f(rid, cid, vid, pid, t0, t1, t2, t3)


NB = 32


def _tcat_body(dts_ref, ats_ref, w16_ref, b16_ref, out_ref):
    n = NB * T
    d = dts_ref[...].reshape(1, n)
    a = ats_ref[...].reshape(1, n)
    w16 = w16_ref[...]
    b16 = b16_ref[...]
    ph_r = w16[0:8, :] * d + b16[0:8, :]
    ph_a = w16[8:16, :] * a + b16[8:16, :]
    ph = jnp.concatenate([ph_r, ph_a], axis=0)
    row = lax.broadcasted_iota(jnp.int32, (16, n), 0)
    out_ref[...] = jnp.where(row % 8 == 0, ph, jnp.sin(ph)).reshape(1, 16, n)


def _tcat(dts, ats, w16, b16):
    return pl.pallas_call(
        _tcat_body,
        grid=(B // NB,),
        in_specs=[
            pl.BlockSpec((1, 1, NB * T), lambda i: (i, 0, 0)),
            pl.BlockSpec((1, 1, NB * T), lambda i: (i, 0, 0)),
            pl.BlockSpec((16, 1), lambda i: (0, 0)),
            pl.BlockSpec((16, 1), lambda i: (0, 0)),
        ],
        out_specs=pl.BlockSpec((1, 16, NB * T), lambda i: (i, 0, 0)),
        out_shape=jax.ShapeDtypeStruct((B // NB, 16, NB * T), jnp.float32),
    )(dts, ats, w16, b16)


def _finish_body(s_ref, tc_ref, pc_ref, time_w_ref, w4_ref,
                 fb_ref, ctok_ref, ctw_ref, g_ref, bb_ref,
                 out_ref):
    n = NB * T
    tcat = tc_ref[...].reshape(16, n)
    m2 = lax.dot_general(w4_ref[...], time_w_ref[...],
                         (((1,), (0,)), ((), ())),
                         preferred_element_type=jnp.float32)
    t_part = lax.dot_general(tcat, m2, (((0,), (1,)), ((), ())),
                             preferred_element_type=jnp.float32)
    scale = 1.0 / math.sqrt(D)
    ev = (s_ref[...].reshape(n, D) + t_part + fb_ref[...]) * scale
    ctx = ctok_ref[...] + lax.dot_general(
        pc_ref[...], ctw_ref[...], (((1,), (1,)), ((), ())),
        preferred_element_type=jnp.float32)
    seq = jnp.concatenate([ctx.reshape(1, NB, D), ev.reshape(T, NB, D)],
                          axis=0)
    mu = jnp.mean(seq, axis=-1, keepdims=True)
    var = jnp.mean((seq - mu) ** 2, axis=-1, keepdims=True)
    out_ref[...] = (seq - mu) * lax.rsqrt(var + 1e-5) * g_ref[...] + bb_ref[...]


def _finish(s3, tc, pc, time_w, final_w, fb, ctok, ctw, ln_g3, ln_b3):
    return pl.pallas_call(
        _finish_body,
        grid=(B // NB,),
        in_specs=[
            pl.BlockSpec((T, NB, D), lambda i: (0, i, 0)),
            pl.BlockSpec((1, 16, NB * T), lambda i: (i, 0, 0)),
            pl.BlockSpec((NB, CTX), lambda i: (i, 0)),
            pl.BlockSpec((D, 16), lambda i: (0, 0)),
            pl.BlockSpec((D, D), lambda i: (0, 4)),
            pl.BlockSpec((1, D), lambda i: (0, 0)),
            pl.BlockSpec((1, D), lambda i: (0, 0)),
            pl.BlockSpec((D, CTX), lambda i: (0, 0)),
            pl.BlockSpec((1, 1, D), lambda i: (0, 0, 0)),
            pl.BlockSpec((1, 1, D), lambda i: (0, 0, 0)),
        ],
        out_specs=pl.BlockSpec((T + 1, NB, D), lambda i: (0, i, 0)),
        out_shape=jax.ShapeDtypeStruct((T + 1, B, D), jnp.float32),
    )(s3, tc, pc, time_w, final_w, fb, ctok, ctw, ln_g3, ln_b3)


def kernel(raw_concept_ids, concept_ids, value_ids, position_ids, delta_ts,
           abs_ts, patient_contexts, raw_tab, conc_tab, val_tab, pos_tab,
           rel_lw, rel_lb, rel_fw, rel_fb, abs_lw, abs_lb, abs_fw, abs_fb,
           time_w, ctx_token, ctx_w, final_w, final_b, ln_g, ln_b):
    rid = raw_concept_ids.T.reshape(BT)
    cid = concept_ids.T.reshape(BT)
    vid = value_ids.T.reshape(BT)
    pid = position_ids.T.reshape(BT)

    t_raw = _transform(raw_tab, final_w, 0)
    t_con = _transform(conc_tab, final_w, 1)
    t_val = _transform(val_tab, final_w, 2)
    t_pos = _transform(pos_tab, final_w, 3)

    s = _gather_sum(rid, cid, vid, pid, t_raw, t_con, t_val, t_pos)

    w16 = jnp.concatenate([rel_lw.reshape(1), rel_fw.reshape(7),
                           abs_lw.reshape(1), abs_fw.reshape(7)]).reshape(16, 1)
    b16 = jnp.concatenate([rel_lb, rel_fb, abs_lb, abs_fb]).reshape(16, 1)

    def tmaj(x):
        return x.reshape(B // NB, NB, T).transpose(0, 2, 1).reshape(
            B // NB, 1, T * NB)

    tc = _tcat(tmaj(delta_ts), tmaj(abs_ts), w16, b16)
    out_t = _finish(s.reshape(T, B, D), tc, patient_contexts, time_w,
                    final_w, final_b.reshape(1, D), ctx_token.reshape(1, D),
                    ctx_w, ln_g.reshape(1, 1, D), ln_b.reshape(1, 1, D))
    return jnp.transpose(out_t, (1, 0, 2))


# speedup vs baseline: 9.1478x; 1.0179x over previous
import functools
import math

import jax
import jax.numpy as jnp
from jax import lax
from jax.experimental import pallas as pl
from jax.experimental.pallas import tpu as pltpu
from jax.experimental.pallas import tpu_sc as plsc

B, T, D = 1024, 200, 128
CTX = 64
BT = B * T

NC, NS = 2, 16
NW = NC * NS
RPW = BT // NW
CHUNK = 80


def _transform_body(tab_ref, w_ref, out_ref):
    out_ref[...] = lax.dot_general(
        tab_ref[...], w_ref[...], (((1,), (1,)), ((), ())),
        preferred_element_type=jnp.float32)


def _transform(tab, final_w, seg):
    v = tab.shape[0]
    rb = 5000 if v % 5000 == 0 else 1000
    return pl.pallas_call(
        _transform_body,
        grid=(v // rb,),
        in_specs=[
            pl.BlockSpec((rb, D), lambda i: (i, 0)),
            pl.BlockSpec((D, D), lambda i, s=seg: (0, s)),
        ],
        out_specs=pl.BlockSpec((rb, D), lambda i: (i, 0)),
        out_shape=jax.ShapeDtypeStruct((v, D), jnp.float32),
    )(tab, final_w)


def _gather_sum_body(rpw, rid, cid, vid, pid, t0, t1, t2, t3, out,
                     idx0, idx1, idx2, idx3,
                     a0, a1, a2, a3, c0, c1, c2, c3,
                     accA, accB, gsemA, gsemB, ssemA, ssemB):
    wid = lax.axis_index("s") * NC + lax.axis_index("c")
    base = wid * rpw
    nch = rpw // CHUNK
    pltpu.sync_copy(rid.at[pl.ds(base, rpw)], idx0)
    pltpu.sync_copy(cid.at[pl.ds(base, rpw)], idx1)
    pltpu.sync_copy(vid.at[pl.ds(base, rpw)], idx2)
    pltpu.sync_copy(pid.at[pl.ds(base, rpw)], idx3)

    def fire(off, bufs, sem):
        pltpu.async_copy(t0.at[idx0.at[pl.ds(off, CHUNK)]], bufs[0], sem)
        pltpu.async_copy(t1.at[idx1.at[pl.ds(off, CHUNK)]], bufs[1], sem)
        pltpu.async_copy(t2.at[idx2.at[pl.ds(off, CHUNK)]], bufs[2], sem)
        pltpu.async_copy(t3.at[idx3.at[pl.ds(off, CHUNK)]], bufs[3], sem)

    def drain(bufs, sem):
        for b in bufs:
            pltpu.make_async_copy(
                t0.at[idx0.at[pl.ds(0, CHUNK)]], b, sem).wait()

    def summed(bufs, acc):
        @plsc.parallel_loop(0, CHUNK, unroll=4)
        def _(r):
            for j in range(D // 16):
                sl = pl.ds(j * 16, 16)
                acc[r, sl] = ((bufs[0][r, sl] + bufs[1][r, sl])
                              + (bufs[2][r, sl] + bufs[3][r, sl]))

    bufsA = (a0, a1, a2, a3)
    bufsB = (c0, c1, c2, c3)
    fire(0, bufsA, gsemA)

    def step(i, carry):
        offa = (2 * i) * CHUNK
        offb = offa + CHUNK
        fire(offb, bufsB, gsemB)
        drain(bufsA, gsemA)

        @pl.when(i > 0)
        def _():
            pltpu.make_async_copy(accA, out.at[pl.ds(base, CHUNK)],
                                  ssemA).wait()
        summed(bufsA, accA)
        pltpu.async_copy(accA, out.at[pl.ds(base + offa, CHUNK)], ssemA)

        @pl.when(i < nch // 2 - 1)
        def _():
            fire(offa + 2 * CHUNK, bufsA, gsemA)
        drain(bufsB, gsemB)

        @pl.when(i > 0)
        def _():
            pltpu.make_async_copy(accB, out.at[pl.ds(base, CHUNK)],
                                  ssemB).wait()
        summed(bufsB, accB)
        pltpu.async_copy(accB, out.at[pl.ds(base + offb, CHUNK)], ssemB)
        return carry

    lax.fori_loop(0, nch // 2, step, 0)
    pltpu.make_async_copy(accA, out.at[pl.ds(base, CHUNK)], ssemA).wait()
    pltpu.make_async_copy(accB, out.at[pl.ds(base, CHUNK)], ssemB).wait()


def _gather_sum(rid, cid, vid, pid, t0, t1, t2, t3):
    rows = rid.shape[0]
    rpw = rows // NW
    mesh = plsc.VectorSubcoreMesh(
        core_axis_name="c", subcore_axis_name="s",
        num_cores=NC, num_subcores=NS)
    buf = pltpu.VMEM((CHUNK, D), jnp.float32)
    acc = pltpu.VMEM((CHUNK, D), jnp.float32)
    f = functools.partial(
        pl.kernel, mesh=mesh,
        out_type=jax.ShapeDtypeStruct((rows, D), jnp.float32),
        scratch_types=[
            pltpu.VMEM((rpw,), jnp.int32),
            pltpu.VMEM((rpw,), jnp.int32),
            pltpu.VMEM((rpw,), jnp.int32),
            pltpu.VMEM((rpw,), jnp.int32),
            buf, buf, buf, buf, buf, buf, buf, buf, acc, acc,
            pltpu.SemaphoreType.DMA,
            pltpu.SemaphoreType.DMA,
            pltpu.SemaphoreType.DMA,
            pltpu.SemaphoreType.DMA,
        ],
    )(functools.partial(_gather_sum_body, rpw))
    return f(rid, cid, vid, pid, t0, t1, t2, t3)


NB = 64


def _tcat_body(dts_ref, ats_ref, w16_ref, b16_ref, out_ref):
    n = NB * T
    d = dts_ref[...].reshape(1, n)
    a = ats_ref[...].reshape(1, n)
    w16 = w16_ref[...]
    b16 = b16_ref[...]
    ph_r = w16[0:8, :] * d + b16[0:8, :]
    ph_a = w16[8:16, :] * a + b16[8:16, :]
    ph = jnp.concatenate([ph_r, ph_a], axis=0)
    row = lax.broadcasted_iota(jnp.int32, (16, n), 0)
    out_ref[...] = jnp.where(row % 8 == 0, ph, jnp.sin(ph)).reshape(1, 16, n)


def _tcat(dts, ats, w16, b16):
    return pl.pallas_call(
        _tcat_body,
        grid=(B // NB,),
        in_specs=[
            pl.BlockSpec((1, 1, NB * T), lambda i: (i, 0, 0)),
            pl.BlockSpec((1, 1, NB * T), lambda i: (i, 0, 0)),
            pl.BlockSpec((16, 1), lambda i: (0, 0)),
            pl.BlockSpec((16, 1), lambda i: (0, 0)),
        ],
        out_specs=pl.BlockSpec((1, 16, NB * T), lambda i: (i, 0, 0)),
        out_shape=jax.ShapeDtypeStruct((B // NB, 16, NB * T), jnp.float32),
    )(dts, ats, w16, b16)


def _finish_body(s_ref, tc_ref, pc_ref, time_w_ref, w4_ref,
                 fb_ref, ctok_ref, ctw_ref, g_ref, bb_ref,
                 out_ref):
    n = NB * T
    tcat = tc_ref[...].reshape(16, n)
    m2 = lax.dot_general(w4_ref[...], time_w_ref[...],
                         (((1,), (0,)), ((), ())),
                         preferred_element_type=jnp.float32)
    t_part = lax.dot_general(tcat, m2, (((0,), (1,)), ((), ())),
                             preferred_element_type=jnp.float32)
    scale = 1.0 / math.sqrt(D)
    ev = (s_ref[...].reshape(n, D) + t_part + fb_ref[...]) * scale
    ctx = ctok_ref[...] + lax.dot_general(
        pc_ref[...], ctw_ref[...], (((1,), (1,)), ((), ())),
        preferred_element_type=jnp.float32)
    seq = jnp.concatenate([ctx.reshape(1, NB, D), ev.reshape(T, NB, D)],
                          axis=0)
    mu = jnp.mean(seq, axis=-1, keepdims=True)
    var = jnp.mean((seq - mu) ** 2, axis=-1, keepdims=True)
    out_ref[...] = (seq - mu) * lax.rsqrt(var + 1e-5) * g_ref[...] + bb_ref[...]


def _finish(s3, tc, pc, time_w, final_w, fb, ctok, ctw, ln_g3, ln_b3):
    return pl.pallas_call(
        _finish_body,
        grid=(B // NB,),
        in_specs=[
            pl.BlockSpec((T, NB, D), lambda i: (0, i, 0)),
            pl.BlockSpec((1, 16, NB * T), lambda i: (i, 0, 0)),
            pl.BlockSpec((NB, CTX), lambda i: (i, 0)),
            pl.BlockSpec((D, 16), lambda i: (0, 0)),
            pl.BlockSpec((D, D), lambda i: (0, 4)),
            pl.BlockSpec((1, D), lambda i: (0, 0)),
            pl.BlockSpec((1, D), lambda i: (0, 0)),
            pl.BlockSpec((D, CTX), lambda i: (0, 0)),
            pl.BlockSpec((1, 1, D), lambda i: (0, 0, 0)),
            pl.BlockSpec((1, 1, D), lambda i: (0, 0, 0)),
        ],
        out_specs=pl.BlockSpec((T + 1, NB, D), lambda i: (0, i, 0)),
        out_shape=jax.ShapeDtypeStruct((T + 1, B, D), jnp.float32),
    )(s3, tc, pc, time_w, final_w, fb, ctok, ctw, ln_g3, ln_b3)


def kernel(raw_concept_ids, concept_ids, value_ids, position_ids, delta_ts,
           abs_ts, patient_contexts, raw_tab, conc_tab, val_tab, pos_tab,
           rel_lw, rel_lb, rel_fw, rel_fb, abs_lw, abs_lb, abs_fw, abs_fb,
           time_w, ctx_token, ctx_w, final_w, final_b, ln_g, ln_b):
    rid = raw_concept_ids.T.reshape(BT)
    cid = concept_ids.T.reshape(BT)
    vid = value_ids.T.reshape(BT)
    pid = position_ids.T.reshape(BT)

    t_raw = _transform(raw_tab, final_w, 0)
    t_con = _transform(conc_tab, final_w, 1)
    t_val = _transform(val_tab, final_w, 2)
    t_pos = _transform(pos_tab, final_w, 3)

    s = _gather_sum(rid, cid, vid, pid, t_raw, t_con, t_val, t_pos)

    w16 = jnp.concatenate([rel_lw.reshape(1), rel_fw.reshape(7),
                           abs_lw.reshape(1), abs_fw.reshape(7)]).reshape(16, 1)
    b16 = jnp.concatenate([rel_lb, rel_fb, abs_lb, abs_fb]).reshape(16, 1)

    def tmaj(x):
        return x.reshape(B // NB, NB, T).transpose(0, 2, 1).reshape(
            B // NB, 1, T * NB)

    tc = _tcat(tmaj(delta_ts), tmaj(abs_ts), w16, b16)
    out_t = _finish(s.reshape(T, B, D), tc, patient_contexts, time_w,
                    final_w, final_b.reshape(1, D), ctx_token.reshape(1, D),
                    ctx_w, ln_g.reshape(1, 1, D), ln_b.reshape(1, 1, D))
    return jnp.transpose(out_t, (1, 0, 2))
